# baseline (speedup 1.0000x reference)
import jax
import jax.numpy as jnp
from jax import lax
from jax.experimental import pallas as pl
from jax.experimental.pallas import tpu as pltpu
from jax.experimental.pallas import tpu_sc as plsc

N = 10240
E = 327680
D_IN = 128
HID = 64
G = 80
NPG = 128

NC = 2
NS = 16
NW = NC * NS
EPW = E // NW
CH = 128
NCHUNK = EPW // CH
ROWS2D = E // CH
NPW = N // NS
GS = 2
NG = NCHUNK // GS

_mesh = plsc.VectorSubcoreMesh(core_axis_name="c", subcore_axis_name="s")


def _deg_body(col_hbm, ew_hbm, out_hbm, col_v, ew_v, zv, deg_sh, dsem):
    cid = lax.axis_index("c")
    sid = lax.axis_index("s")
    wid = sid * NC + cid

    def _zb(i, _):
        zv[pl.ds(i * 16, 16)] = jnp.zeros((16,), jnp.float32)
        return 0
    lax.fori_loop(0, NPW // 16, _zb, 0)
    pltpu.sync_copy(zv, deg_sh.at[pl.ds(sid * NPW, NPW)])

    base = wid * NCHUNK
    pltpu.sync_copy(col_hbm.at[pl.ds(base, NCHUNK)], col_v)
    pltpu.sync_copy(ew_hbm.at[pl.ds(base, NCHUNK)], ew_v)
    plsc.subcore_barrier()

    del dsem
    def _grp(j, _):
        pltpu.sync_copy(ew_v.at[j], deg_sh.at[col_v.at[j]], add=True)
        return 0
    lax.fori_loop(0, NCHUNK, _grp, 0)

    plsc.subcore_barrier()
    pltpu.sync_copy(deg_sh.at[pl.ds(sid * NPW, NPW)],
                    out_hbm.at[cid, pl.ds(sid * NPW, NPW)])


_deg_call = pl.kernel(
    _deg_body,
    out_type=jax.ShapeDtypeStruct((NC, N), jnp.float32),
    mesh=_mesh,
    scratch_types=[
        pltpu.VMEM((NCHUNK, CH), jnp.int32),
        pltpu.VMEM((NCHUNK, CH), jnp.float32),
        pltpu.VMEM((NPW,), jnp.float32),
        pltpu.VMEM_SHARED((N,), jnp.float32),
        pltpu.SemaphoreType.DMA,
    ],
)


def _mm_body(x_ref, w_ref, degp_ref, h2_ref, dis_ref):
    degp = degp_ref[...]
    deg = 1.0 + degp[0] + degp[1]
    dis = jnp.where(deg > 0, lax.rsqrt(jnp.maximum(deg, 1e-12)), 0.0)
    h = jnp.dot(x_ref[...], w_ref[...], preferred_element_type=jnp.float32)
    h2_ref[...] = h * dis
    dis_ref[...] = dis


_BN = 1024

_mm_call = pl.pallas_call(
    _mm_body,
    grid=(N // _BN,),
    in_specs=[
        pl.BlockSpec((_BN, D_IN), lambda i: (i, 0)),
        pl.BlockSpec((D_IN, HID), lambda i: (0, 0)),
        pl.BlockSpec((NC, _BN, 1), lambda i: (0, i, 0)),
    ],
    out_specs=[
        pl.BlockSpec((_BN, HID), lambda i: (i, 0)),
        pl.BlockSpec((_BN, 1), lambda i: (i, 0)),
    ],
    out_shape=[
        jax.ShapeDtypeStruct((N, HID), jnp.float32),
        jax.ShapeDtypeStruct((N, 1), jnp.float32),
    ],
)


def _msg_body(row_hbm, col_hbm, ew_hbm, h2_hbm, out_hbm,
              row_v, col_v, ew_v, rows_v, acc_sh,
              gsem0, gsem1, gsem2, ssem0, ssem1, ssem2):
    cid = lax.axis_index("c")
    sid = lax.axis_index("s")
    wid = sid * NC + cid

    def _zb(i, _):
        for f in range(HID // 16):
            rows_v[0, i, pl.ds(f * 16, 16)] = jnp.zeros((16,), jnp.float32)
        return 0
    lax.fori_loop(0, CH, _zb, 0)
    for i in range(NPW // CH):
        pltpu.sync_copy(rows_v.at[0],
                        acc_sh.at[pl.ds(sid * NPW + i * CH, CH)])

    base = wid * NCHUNK
    pltpu.sync_copy(row_hbm.at[pl.ds(base, NCHUNK)], row_v)
    pltpu.sync_copy(col_hbm.at[pl.ds(base, NCHUNK)], col_v)
    pltpu.sync_copy(ew_hbm.at[pl.ds(base, NCHUNK)], ew_v)
    plsc.subcore_barrier()

    gsems = (gsem0, gsem1, gsem2)
    ssems = (ssem0, ssem1, ssem2)

    def _scale_chunk(j, bi):
        def _gb(g2, _):
            v16 = ew_v[j, pl.ds(g2 * 16, 16)]
            for k in range(16):
                sv = lax.gather(
                    v16, lax.broadcast(k, (16, 1)),
                    lax.GatherDimensionNumbers(
                        offset_dims=(), collapsed_slice_dims=(0,),
                        start_index_map=(0,)),
                    (1,), mode=lax.GatherScatterMode.PROMISE_IN_BOUNDS)
                e = g2 * 16 + k
                for f in range(HID // 16):
                    sl = pl.ds(f * 16, 16)
                    rows_v[bi, e, sl] = rows_v[bi, e, sl] * sv
            return 0
        lax.fori_loop(0, CH // 16, _gb, 0)

    def _gfire(t, s):
        for u in range(GS):
            pltpu.make_async_copy(h2_hbm.at[row_v.at[t * GS + u]],
                                  rows_v.at[s * GS + u], gsems[s]).start()

    def _gdrain(s):
        for _u in range(GS):
            pltpu.make_async_copy(h2_hbm.at[row_v.at[0]],
                                  rows_v.at[0], gsems[s]).wait()

    def _sfire(t, s):
        for u in range(GS):
            pltpu.make_async_copy(rows_v.at[s * GS + u],
                                  acc_sh.at[col_v.at[t * GS + u]],
                                  ssems[s]).start(add=True)

    def _sdrain(s):
        for _u in range(GS):
            pltpu.make_async_copy(rows_v.at[0],
                                  acc_sh.at[col_v.at[0]], ssems[s]).wait()

    _gfire(0, 0)

    def _pipe(t3, _):
        for u3 in range(3):
            t = t3 * 3 + u3
            s = u3
            sn = (u3 + 1) % 3

            @pl.when(jnp.logical_and(t >= 2, t < NG + 2))
            def _():
                _sdrain(sn)

            @pl.when(t + 1 < NG)
            def _():
                _gfire(t + 1, sn)

            @pl.when(t < NG)
            def _():
                _gdrain(s)
                for u in range(GS):
                    _scale_chunk(t * GS + u, s * GS + u)
                _sfire(t, s)
        return 0

    lax.fori_loop(0, (NG + 3 + 2) // 3, _pipe, 0)

    plsc.subcore_barrier()
    pltpu.sync_copy(acc_sh.at[pl.ds(sid * NPW, NPW)],
                    out_hbm.at[cid, pl.ds(sid * NPW, NPW)])


_msg_call = pl.kernel(
    _msg_body,
    out_type=jax.ShapeDtypeStruct((NC, N, HID), jnp.float32),
    mesh=_mesh,
    scratch_types=[
        pltpu.VMEM((NCHUNK, CH), jnp.int32),
        pltpu.VMEM((NCHUNK, CH), jnp.int32),
        pltpu.VMEM((NCHUNK, CH), jnp.float32),
        pltpu.VMEM((3 * GS, CH, HID), jnp.float32),
        pltpu.VMEM_SHARED((N, HID), jnp.float32),
        pltpu.SemaphoreType.DMA,
        pltpu.SemaphoreType.DMA,
        pltpu.SemaphoreType.DMA,
        pltpu.SemaphoreType.DMA,
        pltpu.SemaphoreType.DMA,
        pltpu.SemaphoreType.DMA,
    ],
    compiler_params=pltpu.CompilerParams(use_tc_tiling_on_sc=False),
)


def _head_body(accp_ref, h2_ref, dis_ref, b1_ref, wlt_ref, sel_ref, bl_ref,
               out_ref):
    a = accp_ref[...]
    nodes = dis_ref[...] * (a[0] + a[1] + h2_ref[...]) + b1_ref[...]
    nodes = jnp.maximum(nodes, 0.0)
    p = nodes * wlt_ref[...]
    s = jnp.dot(sel_ref[...], p, preferred_element_type=jnp.float32)
    logit = jnp.sum(s, axis=1, keepdims=True) + bl_ref[...]
    out_ref[...] = jax.nn.sigmoid(logit)


_GB = _BN // NPG

_head_call = pl.pallas_call(
    _head_body,
    grid=(N // _BN,),
    in_specs=[
        pl.BlockSpec((NC, _BN, HID), lambda i: (0, i, 0)),
        pl.BlockSpec((_BN, HID), lambda i: (i, 0)),
        pl.BlockSpec((_BN, 1), lambda i: (i, 0)),
        pl.BlockSpec((1, HID), lambda i: (0, 0)),
        pl.BlockSpec((_BN, HID), lambda i: (0, 0)),
        pl.BlockSpec((_GB, _BN), lambda i: (0, 0)),
        pl.BlockSpec((1, 1), lambda i: (0, 0)),
    ],
    out_specs=pl.BlockSpec((_GB, 1), lambda i: (i, 0)),
    out_shape=jax.ShapeDtypeStruct((G, 1), jnp.float32),
)


@jax.jit
def kernel(x, edge_index, edge_weight, batch, W1, b1, W_lin, b_lin):
    del batch
    row2d = edge_index[0].reshape(ROWS2D, CH)
    col2d = edge_index[1].reshape(ROWS2D, CH)
    ew2d = edge_weight.astype(jnp.float32).reshape(ROWS2D, CH)

    degp = _deg_call(col2d, ew2d)
    h2, dis = _mm_call(x, W1, degp.reshape(NC, N, 1))
    accp = _msg_call(row2d, col2d, ew2d, h2)

    wlt = jnp.tile(W_lin.reshape(NPG, HID), (_GB, 1))
    sel = (jnp.arange(_BN, dtype=jnp.int32)[None, :] // NPG
           == jnp.arange(_GB, dtype=jnp.int32)[:, None]).astype(jnp.float32)
    out = _head_call(accp, h2, dis, b1.reshape(1, HID), wlt, sel,
                     b_lin.reshape(1, 1))
    return out


# speedup vs baseline: 1.0427x; 1.0427x over previous
import jax
import jax.numpy as jnp
from jax import lax
from jax.experimental import pallas as pl
from jax.experimental.pallas import tpu as pltpu
from jax.experimental.pallas import tpu_sc as plsc

N = 10240
E = 327680
D_IN = 128
HID = 64
G = 80
NPG = 128

NC = 2
NS = 16
NW = NC * NS
EPW = E // NW
CH = 128
NCHUNK = EPW // CH
ROWS2D = E // CH
NPW = N // NS
NSET = 6
GPF = 3

_mesh = plsc.VectorSubcoreMesh(core_axis_name="c", subcore_axis_name="s")


def _deg_body(col_hbm, ew_hbm, out_hbm, col_v, ew_v, zv, deg_sh, dsem):
    cid = lax.axis_index("c")
    sid = lax.axis_index("s")
    wid = sid * NC + cid

    def _zb(i, _):
        zv[pl.ds(i * 16, 16)] = jnp.zeros((16,), jnp.float32)
        return 0
    lax.fori_loop(0, NPW // 16, _zb, 0)
    pltpu.sync_copy(zv, deg_sh.at[pl.ds(sid * NPW, NPW)])

    base = wid * NCHUNK
    pltpu.sync_copy(col_hbm.at[pl.ds(base, NCHUNK)], col_v)
    pltpu.sync_copy(ew_hbm.at[pl.ds(base, NCHUNK)], ew_v)
    plsc.subcore_barrier()

    del dsem
    def _grp(j, _):
        pltpu.sync_copy(ew_v.at[j], deg_sh.at[col_v.at[j]], add=True)
        return 0
    lax.fori_loop(0, NCHUNK, _grp, 0)

    plsc.subcore_barrier()
    pltpu.sync_copy(deg_sh.at[pl.ds(sid * NPW, NPW)],
                    out_hbm.at[cid, pl.ds(sid * NPW, NPW)])


_deg_call = pl.kernel(
    _deg_body,
    out_type=jax.ShapeDtypeStruct((NC, N), jnp.float32),
    mesh=_mesh,
    scratch_types=[
        pltpu.VMEM((NCHUNK, CH), jnp.int32),
        pltpu.VMEM((NCHUNK, CH), jnp.float32),
        pltpu.VMEM((NPW,), jnp.float32),
        pltpu.VMEM_SHARED((N,), jnp.float32),
        pltpu.SemaphoreType.DMA,
    ],
)


def _mm_body(x_ref, w_ref, degp_ref, h2_ref, dis_ref):
    degp = degp_ref[...]
    deg = 1.0 + degp[0] + degp[1]
    dis = jnp.where(deg > 0, lax.rsqrt(jnp.maximum(deg, 1e-12)), 0.0)
    h = jnp.dot(x_ref[...], w_ref[...], preferred_element_type=jnp.float32)
    h2_ref[...] = h * dis
    dis_ref[...] = dis


_BN = 1024

_mm_call = pl.pallas_call(
    _mm_body,
    grid=(N // _BN,),
    in_specs=[
        pl.BlockSpec((_BN, D_IN), lambda i: (i, 0)),
        pl.BlockSpec((D_IN, HID), lambda i: (0, 0)),
        pl.BlockSpec((NC, _BN, 1), lambda i: (0, i, 0)),
    ],
    out_specs=[
        pl.BlockSpec((_BN, HID), lambda i: (i, 0)),
        pl.BlockSpec((_BN, 1), lambda i: (i, 0)),
    ],
    out_shape=[
        jax.ShapeDtypeStruct((N, HID), jnp.float32),
        jax.ShapeDtypeStruct((N, 1), jnp.float32),
    ],
)


def _msg_body(row_hbm, col_hbm, ew_hbm, h2_hbm, out_hbm,
              row_v, col_v, ew_v, rows_v, acc_sh,
              gsem0, gsem1, gsem2, gsem3, gsem4, gsem5,
              ssem0, ssem1, ssem2, ssem3, ssem4, ssem5):
    cid = lax.axis_index("c")
    sid = lax.axis_index("s")
    wid = sid * NC + cid

    def _zb(i, _):
        for f in range(HID // 16):
            rows_v[0, i, pl.ds(f * 16, 16)] = jnp.zeros((16,), jnp.float32)
        return 0
    lax.fori_loop(0, CH, _zb, 0)
    for i in range(NPW // CH):
        pltpu.sync_copy(rows_v.at[0],
                        acc_sh.at[pl.ds(sid * NPW + i * CH, CH)])

    base = wid * NCHUNK
    pltpu.sync_copy(row_hbm.at[pl.ds(base, NCHUNK)], row_v)
    pltpu.sync_copy(col_hbm.at[pl.ds(base, NCHUNK)], col_v)
    pltpu.sync_copy(ew_hbm.at[pl.ds(base, NCHUNK)], ew_v)
    plsc.subcore_barrier()

    gsems = (gsem0, gsem1, gsem2, gsem3, gsem4, gsem5)
    ssems = (ssem0, ssem1, ssem2, ssem3, ssem4, ssem5)

    def _scale_chunk(j, bi):
        def _gb(g2, _):
            v16 = ew_v[j, pl.ds(g2 * 16, 16)]
            for k in range(16):
                sv = lax.gather(
                    v16, lax.broadcast(k, (16, 1)),
                    lax.GatherDimensionNumbers(
                        offset_dims=(), collapsed_slice_dims=(0,),
                        start_index_map=(0,)),
                    (1,), mode=lax.GatherScatterMode.PROMISE_IN_BOUNDS)
                e = g2 * 16 + k
                for f in range(HID // 16):
                    sl = pl.ds(f * 16, 16)
                    rows_v[bi, e, sl] = rows_v[bi, e, sl] * sv
            return 0
        lax.fori_loop(0, CH // 16, _gb, 0)

    def _gfire(t, s):
        pltpu.make_async_copy(h2_hbm.at[row_v.at[t]],
                              rows_v.at[s], gsems[s]).start()

    def _gdrain(s):
        pltpu.make_async_copy(h2_hbm.at[row_v.at[0]],
                              rows_v.at[0], gsems[s]).wait()

    def _sfire(t, s):
        pltpu.make_async_copy(rows_v.at[s],
                              acc_sh.at[col_v.at[t]], ssems[0]).start(add=True)

    def _sdrain(s):
        pltpu.make_async_copy(rows_v.at[0],
                              acc_sh.at[col_v.at[0]], ssems[s]).wait()

    for p in range(GPF):
        _gfire(p, p)

    def _pipe(t6, _):
        for u6 in range(NSET):
            t = t6 * NSET + u6
            s = u6
            sg = (u6 + GPF) % NSET

            @pl.when(t + GPF < NCHUNK)
            def _():
                _gfire(t + GPF, sg)

            @pl.when(t < NCHUNK)
            def _():
                _gdrain(s)
                _scale_chunk(t, s)

            @pl.when(jnp.logical_and(t >= 1, t < NCHUNK + 1))
            def _():
                _sdrain(0)

            @pl.when(t < NCHUNK)
            def _():
                _sfire(t, s)
        return 0

    lax.fori_loop(0, (NCHUNK + 1 + NSET - 1) // NSET, _pipe, 0)

    plsc.subcore_barrier()
    pltpu.sync_copy(acc_sh.at[pl.ds(sid * NPW, NPW)],
                    out_hbm.at[cid, pl.ds(sid * NPW, NPW)])


_msg_call = pl.kernel(
    _msg_body,
    out_type=jax.ShapeDtypeStruct((NC, N, HID), jnp.float32),
    mesh=_mesh,
    scratch_types=[
        pltpu.VMEM((NCHUNK, CH), jnp.int32),
        pltpu.VMEM((NCHUNK, CH), jnp.int32),
        pltpu.VMEM((NCHUNK, CH), jnp.float32),
        pltpu.VMEM((NSET, CH, HID), jnp.float32),
        pltpu.VMEM_SHARED((N, HID), jnp.float32),
    ] + [pltpu.SemaphoreType.DMA] * 12,
    compiler_params=pltpu.CompilerParams(use_tc_tiling_on_sc=False),
)


def _head_body(accp_ref, h2_ref, dis_ref, b1_ref, wlt_ref, sel_ref, bl_ref,
               out_ref):
    a = accp_ref[...]
    nodes = dis_ref[...] * (a[0] + a[1] + h2_ref[...]) + b1_ref[...]
    nodes = jnp.maximum(nodes, 0.0)
    p = nodes * wlt_ref[...]
    s = jnp.dot(sel_ref[...], p, preferred_element_type=jnp.float32)
    logit = jnp.sum(s, axis=1, keepdims=True) + bl_ref[...]
    out_ref[...] = jax.nn.sigmoid(logit)


_GB = _BN // NPG

_head_call = pl.pallas_call(
    _head_body,
    grid=(N // _BN,),
    in_specs=[
        pl.BlockSpec((NC, _BN, HID), lambda i: (0, i, 0)),
        pl.BlockSpec((_BN, HID), lambda i: (i, 0)),
        pl.BlockSpec((_BN, 1), lambda i: (i, 0)),
        pl.BlockSpec((1, HID), lambda i: (0, 0)),
        pl.BlockSpec((_BN, HID), lambda i: (0, 0)),
        pl.BlockSpec((_GB, _BN), lambda i: (0, 0)),
        pl.BlockSpec((1, 1), lambda i: (0, 0)),
    ],
    out_specs=pl.BlockSpec((_GB, 1), lambda i: (i, 0)),
    out_shape=jax.ShapeDtypeStruct((G, 1), jnp.float32),
)


@jax.jit
def kernel(x, edge_index, edge_weight, batch, W1, b1, W_lin, b_lin):
    del batch
    row2d = edge_index[0].reshape(ROWS2D, CH)
    col2d = edge_index[1].reshape(ROWS2D, CH)
    ew2d = edge_weight.astype(jnp.float32).reshape(ROWS2D, CH)

    degp = _deg_call(col2d, ew2d)
    h2, dis = _mm_call(x, W1, degp.reshape(NC, N, 1))
    accp = _msg_call(row2d, col2d, ew2d, h2)

    wlt = jnp.tile(W_lin.reshape(NPG, HID), (_GB, 1))
    sel = (jnp.arange(_BN, dtype=jnp.int32)[None, :] // NPG
           == jnp.arange(_GB, dtype=jnp.int32)[:, None]).astype(jnp.float32)
    out = _head_call(accp, h2, dis, b1.reshape(1, HID), wlt, sel,
                     b_lin.reshape(1, 1))
    return out


# speedup vs baseline: 1.2954x; 1.2424x over previous
import jax
import jax.numpy as jnp
from jax import lax
from jax.experimental import pallas as pl
from jax.experimental.pallas import tpu as pltpu
from jax.experimental.pallas import tpu_sc as plsc

N = 10240
E = 327680
D_IN = 128
HID = 64
G = 80
NPG = 128

NC = 2
NS = 16
NW = NC * NS
EPW = E // NW
CH = 128
NCHUNK = EPW // CH
ROWS2D = E // CH
NPW = N // NS
NSET = 6
GPF = 3

_mesh = plsc.VectorSubcoreMesh(core_axis_name="c", subcore_axis_name="s")


def _deg_body(col_hbm, ew_hbm, out_hbm, col_v, ew_v, zv, deg_sh, dsem):
    cid = lax.axis_index("c")
    sid = lax.axis_index("s")
    wid = sid * NC + cid

    def _zb(i, _):
        zv[pl.ds(i * 16, 16)] = jnp.zeros((16,), jnp.float32)
        return 0
    lax.fori_loop(0, NPW // 16, _zb, 0)
    pltpu.sync_copy(zv, deg_sh.at[pl.ds(sid * NPW, NPW)])

    base = wid * NCHUNK
    pltpu.sync_copy(col_hbm.at[pl.ds(base, NCHUNK)], col_v)
    pltpu.sync_copy(ew_hbm.at[pl.ds(base, NCHUNK)], ew_v)
    plsc.subcore_barrier()

    del dsem
    def _grp(j, _):
        pltpu.sync_copy(ew_v.at[j], deg_sh.at[col_v.at[j]], add=True)
        return 0
    lax.fori_loop(0, NCHUNK, _grp, 0)

    plsc.subcore_barrier()
    pltpu.sync_copy(deg_sh.at[pl.ds(sid * NPW, NPW)],
                    out_hbm.at[cid, pl.ds(sid * NPW, NPW)])


_deg_call = pl.kernel(
    _deg_body,
    out_type=jax.ShapeDtypeStruct((NC, N), jnp.float32),
    mesh=_mesh,
    scratch_types=[
        pltpu.VMEM((NCHUNK, CH), jnp.int32),
        pltpu.VMEM((NCHUNK, CH), jnp.float32),
        pltpu.VMEM((NPW,), jnp.float32),
        pltpu.VMEM_SHARED((N,), jnp.float32),
        pltpu.SemaphoreType.DMA,
    ],
)


def _mm_body(x_ref, w_ref, degp_ref, h2_ref, dis_ref):
    degp = degp_ref[...]
    deg = 1.0 + degp[0] + degp[1]
    dis = jnp.where(deg > 0, lax.rsqrt(jnp.maximum(deg, 1e-12)), 0.0)
    h = jnp.dot(x_ref[...], w_ref[...], preferred_element_type=jnp.float32)
    h2_ref[...] = h * dis
    dis_ref[...] = dis


_BN = 1024

_mm_call = pl.pallas_call(
    _mm_body,
    grid=(N // _BN,),
    in_specs=[
        pl.BlockSpec((_BN, D_IN), lambda i: (i, 0)),
        pl.BlockSpec((D_IN, HID), lambda i: (0, 0)),
        pl.BlockSpec((NC, _BN, 1), lambda i: (0, i, 0)),
    ],
    out_specs=[
        pl.BlockSpec((_BN, HID), lambda i: (i, 0)),
        pl.BlockSpec((_BN, 1), lambda i: (i, 0)),
    ],
    out_shape=[
        jax.ShapeDtypeStruct((N, HID), jnp.float32),
        jax.ShapeDtypeStruct((N, 1), jnp.float32),
    ],
)


def _msg_body(row_hbm, col_hbm, ew_hbm, h2_hbm, out_hbm,
              row_v, col_v, ew_v, rows_v, acc_sh,
              gsem0, gsem1, gsem2, gsem3, gsem4, gsem5,
              ssem0, ssem1, ssem2, ssem3, ssem4, ssem5):
    cid = lax.axis_index("c")
    sid = lax.axis_index("s")
    wid = sid * NC + cid

    def _zb(i, _):
        for f in range(HID // 16):
            rows_v[0, i, pl.ds(f * 16, 16)] = jnp.zeros((16,), jnp.float32)
        return 0
    lax.fori_loop(0, CH, _zb, 0)
    for i in range(NPW // CH):
        pltpu.sync_copy(rows_v.at[0],
                        acc_sh.at[pl.ds(sid * NPW + i * CH, CH)])

    base = wid * NCHUNK
    pltpu.sync_copy(row_hbm.at[pl.ds(base, NCHUNK)], row_v)
    pltpu.sync_copy(col_hbm.at[pl.ds(base, NCHUNK)], col_v)
    pltpu.sync_copy(ew_hbm.at[pl.ds(base, NCHUNK)], ew_v)
    plsc.subcore_barrier()

    gsems = (gsem0, gsem1, gsem2, gsem3, gsem4, gsem5)
    ssems = (ssem0, ssem1, ssem2, ssem3, ssem4, ssem5)

    def _scale_chunk(j, bi):
        @plsc.parallel_loop(0, CH // 16, unroll=2)
        def _gb(g2):
            v16 = ew_v[j, pl.ds(g2 * 16, 16)]
            for k in range(16):
                sv = lax.gather(
                    v16, lax.broadcast(k, (16, 1)),
                    lax.GatherDimensionNumbers(
                        offset_dims=(), collapsed_slice_dims=(0,),
                        start_index_map=(0,)),
                    (1,), mode=lax.GatherScatterMode.PROMISE_IN_BOUNDS)
                e = g2 * 16 + k
                for f in range(HID // 16):
                    sl = pl.ds(f * 16, 16)
                    rows_v[bi, e, sl] = rows_v[bi, e, sl] * sv

    def _gfire(t, s):
        pltpu.make_async_copy(h2_hbm.at[row_v.at[t]],
                              rows_v.at[s], gsems[s]).start()

    def _gdrain(s):
        pltpu.make_async_copy(h2_hbm.at[row_v.at[0]],
                              rows_v.at[0], gsems[s]).wait()

    def _sfire(t, s):
        pltpu.make_async_copy(rows_v.at[s],
                              acc_sh.at[col_v.at[t]], ssems[0]).start(add=True)

    def _sdrain(s):
        pltpu.make_async_copy(rows_v.at[0],
                              acc_sh.at[col_v.at[0]], ssems[s]).wait()

    for p in range(GPF):
        _gfire(p, p)

    def _pipe(t6, _):
        for u6 in range(NSET):
            t = t6 * NSET + u6
            s = u6
            sg = (u6 + GPF) % NSET

            @pl.when(t + GPF < NCHUNK)
            def _():
                _gfire(t + GPF, sg)

            @pl.when(t < NCHUNK)
            def _():
                _gdrain(s)
                _scale_chunk(t, s)

            @pl.when(jnp.logical_and(t >= 1, t < NCHUNK + 1))
            def _():
                _sdrain(0)

            @pl.when(t < NCHUNK)
            def _():
                _sfire(t, s)
        return 0

    lax.fori_loop(0, (NCHUNK + 1 + NSET - 1) // NSET, _pipe, 0)

    plsc.subcore_barrier()
    pltpu.sync_copy(acc_sh.at[pl.ds(sid * NPW, NPW)],
                    out_hbm.at[cid, pl.ds(sid * NPW, NPW)])


_msg_call = pl.kernel(
    _msg_body,
    out_type=jax.ShapeDtypeStruct((NC, N, HID), jnp.float32),
    mesh=_mesh,
    scratch_types=[
        pltpu.VMEM((NCHUNK, CH), jnp.int32),
        pltpu.VMEM((NCHUNK, CH), jnp.int32),
        pltpu.VMEM((NCHUNK, CH), jnp.float32),
        pltpu.VMEM((NSET, CH, HID), jnp.float32),
        pltpu.VMEM_SHARED((N, HID), jnp.float32),
    ] + [pltpu.SemaphoreType.DMA] * 12,
    compiler_params=pltpu.CompilerParams(use_tc_tiling_on_sc=False),
)


def _head_body(accp_ref, h2_ref, dis_ref, b1_ref, wlt_ref, sel_ref, bl_ref,
               out_ref):
    a = accp_ref[...]
    nodes = dis_ref[...] * (a[0] + a[1] + h2_ref[...]) + b1_ref[...]
    nodes = jnp.maximum(nodes, 0.0)
    p = nodes * wlt_ref[...]
    s = jnp.dot(sel_ref[...], p, preferred_element_type=jnp.float32)
    logit = jnp.sum(s, axis=1, keepdims=True) + bl_ref[...]
    out_ref[...] = jax.nn.sigmoid(logit)


_GB = _BN // NPG

_head_call = pl.pallas_call(
    _head_body,
    grid=(N // _BN,),
    in_specs=[
        pl.BlockSpec((NC, _BN, HID), lambda i: (0, i, 0)),
        pl.BlockSpec((_BN, HID), lambda i: (i, 0)),
        pl.BlockSpec((_BN, 1), lambda i: (i, 0)),
        pl.BlockSpec((1, HID), lambda i: (0, 0)),
        pl.BlockSpec((_BN, HID), lambda i: (0, 0)),
        pl.BlockSpec((_GB, _BN), lambda i: (0, 0)),
        pl.BlockSpec((1, 1), lambda i: (0, 0)),
    ],
    out_specs=pl.BlockSpec((_GB, 1), lambda i: (i, 0)),
    out_shape=jax.ShapeDtypeStruct((G, 1), jnp.float32),
)


@jax.jit
def kernel(x, edge_index, edge_weight, batch, W1, b1, W_lin, b_lin):
    del batch
    row2d = edge_index[0].reshape(ROWS2D, CH)
    col2d = edge_index[1].reshape(ROWS2D, CH)
    ew2d = edge_weight.astype(jnp.float32).reshape(ROWS2D, CH)

    degp = _deg_call(col2d, ew2d)
    h2, dis = _mm_call(x, W1, degp.reshape(NC, N, 1))
    accp = _msg_call(row2d, col2d, ew2d, h2)

    wlt = jnp.tile(W_lin.reshape(NPG, HID), (_GB, 1))
    sel = (jnp.arange(_BN, dtype=jnp.int32)[None, :] // NPG
           == jnp.arange(_GB, dtype=jnp.int32)[:, None]).astype(jnp.float32)
    out = _head_call(accp, h2, dis, b1.reshape(1, HID), wlt, sel,
                     b_lin.reshape(1, 1))
    return out
